# Initial kernel scaffold; baseline (speedup 1.0000x reference)
import jax
import jax.numpy as jnp
from jax.experimental import pallas as pl


def kernel(x0, x1, x2, x3, query_feat, query_roi, W_off, b_off, W_pg, b_pg, W_out, b_out, ln_g, ln_b):
    raise NotImplementedError("write your pallas kernel here")


# speedup vs baseline: 1.0031x; 1.0031x over previous
import jax
import jax.numpy as jnp
from jax.experimental import pallas as pl

B, N = 2, 300
CONTENT_DIM = 256
FEAT_CH = 256
IN_POINTS = 32
OUT_POINTS = 128
N_GROUPS = 4
STRIDES = [4.0, 8.0, 16.0, 32.0]
IMG = 512


def _ln_kernel(q_ref, g_ref, b_ref, o_ref):
    q = q_ref[...]
    m = jnp.mean(q, axis=-1, keepdims=True)
    v = jnp.mean((q - m) ** 2, axis=-1, keepdims=True)
    o_ref[...] = (q - m) * jax.lax.rsqrt(v + 1e-5) * g_ref[...] + b_ref[...]


def _final_ln(q, ln_g, ln_b):
    Bq, Nq, C = q.shape
    out = pl.pallas_call(
        _ln_kernel,
        out_shape=jax.ShapeDtypeStruct((Bq * Nq, C), jnp.float32),
    )(q.reshape(Bq * Nq, C), ln_g.reshape(1, C), ln_b.reshape(1, C))
    return out.reshape(Bq, Nq, C)


def _layer_norm_2d(x, eps=1e-5):
    m = x.mean(axis=(-2, -1), keepdims=True)
    v = ((x - m) ** 2).mean(axis=(-2, -1), keepdims=True)
    return (x - m) / jnp.sqrt(v + eps)


def _make_sample_points(offset, xyzr):
    Bq, L, _ = offset.shape
    num_group = N_GROUPS * IN_POINTS
    off = offset.reshape(Bq, L, 1, num_group, 3)
    roi_cc = xyzr[..., :2]
    scale = 2.0 ** xyzr[..., 2:3]
    ratio = 2.0 ** jnp.concatenate([xyzr[..., 3:4] * -0.5, xyzr[..., 3:4] * 0.5], axis=-1)
    roi_wh = scale * ratio
    roi_lvl = xyzr[..., 2:3].reshape(Bq, L, 1, 1, 1)
    offset_yx = off[..., :2] * roi_wh.reshape(Bq, L, 1, 1, 2)
    sample_yx = roi_cc.reshape(Bq, L, 1, 1, 2) + offset_yx
    sample_lvl = roi_lvl + off[..., 2:3]
    return jnp.concatenate([sample_yx, sample_lvl], axis=-1)


def _grid_sample(value, grid):
    BG, C, H, W = value.shape
    x = ((grid[..., 0] + 1.0) * W - 1.0) / 2.0
    y = ((grid[..., 1] + 1.0) * H - 1.0) / 2.0
    x0 = jnp.floor(x)
    y0 = jnp.floor(y)
    flat = value.reshape(BG, C, H * W)
    qshape = x.shape[1:]

    def corner(xi, yi):
        valid = (xi >= 0) & (xi <= W - 1) & (yi >= 0) & (yi <= H - 1)
        xc = jnp.clip(xi, 0, W - 1).astype(jnp.int32)
        yc = jnp.clip(yi, 0, H - 1).astype(jnp.int32)
        idx = (yc * W + xc).reshape(BG, -1)
        g = jax.vmap(lambda v, i: v[:, i])(flat, idx)
        g = g.reshape((BG, C) + qshape)
        return g * valid.reshape((BG, 1) + qshape).astype(value.dtype)

    w00 = (x0 + 1.0 - x) * (y0 + 1.0 - y)
    w10 = (x - x0) * (y0 + 1.0 - y)
    w01 = (x0 + 1.0 - x) * (y - y0)
    w11 = (x - x0) * (y - y0)
    out = (corner(x0, y0) * w00[:, None]
           + corner(x0 + 1.0, y0) * w10[:, None]
           + corner(x0, y0 + 1.0) * w01[:, None]
           + corner(x0 + 1.0, y0 + 1.0) * w11[:, None])
    return out


def _sampling_each_level(sample_points, value, weight, n_points):
    Bq, n_queries = sample_points.shape[0], sample_points.shape[1]
    n_groups_points = sample_points.shape[2] * sample_points.shape[3]
    B2, C_feat, H_feat, W_feat = value.shape
    n_groups = n_groups_points // n_points
    n_channels = C_feat // n_groups
    sp = sample_points.reshape(Bq, n_queries, n_groups, n_points, 2)
    sp = jnp.transpose(sp, (0, 2, 1, 3, 4)).reshape(Bq * n_groups, n_queries, n_points, 2)
    sp = sp * 2.0 - 1.0
    val = value.reshape(Bq * n_groups, n_channels, H_feat, W_feat)
    out = _grid_sample(val, sp)
    w = weight.reshape(Bq, n_queries, n_groups, n_points)
    w = jnp.transpose(w, (0, 2, 1, 3)).reshape(Bq * n_groups, 1, n_queries, n_points)
    out = out * w
    out = out.reshape(Bq, n_groups, n_channels, n_queries, n_points)
    return jnp.transpose(out, (0, 3, 1, 4, 2))


def _sampling_3d(sample_points, mlvl_values, featmap_strides, n_points):
    Bq, n_queries = sample_points.shape[0], sample_points.shape[1]
    C_feat = mlvl_values[0].shape[1]
    n_groups = (sample_points.shape[2] * sample_points.shape[3]) // n_points
    xy = sample_points[..., 0:2]
    z = sample_points[..., 2]
    grid = jnp.log2(jnp.asarray(featmap_strides, jnp.float32))
    l2 = -jnp.abs(((z[..., None] - grid) ** 2) / 2.0)
    lvl_w = jax.nn.softmax(l2, axis=-1)
    out = jnp.zeros((Bq, n_queries, n_groups, n_points, FEAT_CH // n_groups), jnp.float32)
    for i in range(len(mlvl_values)):
        value = mlvl_values[i]
        stride = featmap_strides[i]
        mapping_size = jnp.asarray([value.shape[3], value.shape[2]], jnp.float32).reshape(1, 1, 1, 1, 2) * stride
        nxy = xy / mapping_size
        out = out + _sampling_each_level(nxy, value, lvl_w[..., i], n_points)
    return out


def _adaptive_mixing(x, query, W_pg, b_pg, W_out, b_out):
    Bq, Nq, G, P, C = x.shape
    eff_out = FEAT_CH // N_GROUPS
    m_params = C * eff_out
    params = query @ W_pg + b_pg
    params = params.reshape(Bq * Nq, G, -1)
    out = x.reshape(Bq * Nq, G, P, C)
    M = params[..., :m_params].reshape(Bq * Nq, G, C, eff_out)
    S = params[..., m_params:].reshape(Bq * Nq, G, OUT_POINTS, IN_POINTS)
    out = jnp.matmul(out, M)
    out = jax.nn.relu(_layer_norm_2d(out))
    out = jnp.matmul(S, out)
    out = jax.nn.relu(_layer_norm_2d(out))
    out = out.reshape(Bq, Nq, -1)
    out = out @ W_out + b_out
    return query + out


def kernel(x0, x1, x2, x3, query_feat, query_roi, W_off, b_off, W_pg, b_pg, W_out, b_out, ln_g, ln_b):
    offset = query_feat @ W_off + b_off
    sp = _make_sample_points(offset, query_roi)
    sampled = _sampling_3d(sp, [x0, x1, x2, x3], STRIDES, IN_POINTS)
    q = _adaptive_mixing(sampled, query_feat, W_pg, b_pg, W_out, b_out)
    return _final_ln(q, ln_g, ln_b)


# speedup vs baseline: 7.9095x; 7.8850x over previous
import functools

import jax
import jax.numpy as jnp
from jax import lax
from jax.experimental import pallas as pl
from jax.experimental.pallas import tpu as pltpu
from jax.experimental.pallas import tpu_sc as plsc

B, N = 2, 300
CONTENT_DIM = 256
FEAT_CH = 256
IN_POINTS = 32
OUT_POINTS = 128
N_GROUPS = 4
STRIDES = [4.0, 8.0, 16.0, 32.0]
IMG = 512

_NC, _NS, _LANES = 2, 16, 16
_NW = _NC * _NS
_S_TOT = B * N * N_GROUPS * IN_POINTS
_PER_W = _S_TOT // _NW
_K = 96
_NCHUNK = _PER_W // _K
_NT = 16
_CG = FEAT_CH // N_GROUPS

_SIZES = [IMG // int(s) for s in STRIDES]
_LVL_BASE = []
_acc = 0
for _hw in _SIZES:
    _LVL_BASE.append(_acc)
    _acc += B * N_GROUPS * _hw * _hw
_ROWS_TOT = _acc


def _sc_gather_fn():
    mesh = plsc.VectorSubcoreMesh(
        core_axis_name="c", subcore_axis_name="s",
        num_cores=_NC, num_subcores=_NS)

    @functools.partial(
        pl.kernel,
        out_type=jax.ShapeDtypeStruct((_S_TOT, _CG), jnp.float32),
        mesh=mesh,
        scratch_types=[
            pltpu.VMEM((_NT, _K, _CG), jnp.float32),
            pltpu.VMEM((_K, _CG), jnp.float32),
            pltpu.VMEM((_NT, _K), jnp.int32),
            pltpu.VMEM((_K, _NT), jnp.float32),
            pltpu.SemaphoreType.DMA,
        ],
        compiler_params=pltpu.CompilerParams(use_tc_tiling_on_sc=False),
    )
    def sc_gather(tall, idxs, cws, out, buf, obuf, idxb, cwb, sem):
        wid = lax.axis_index("s") * _NC + lax.axis_index("c")

        def chunk(j, carry):
            c = wid * _NCHUNK + j
            pltpu.sync_copy(idxs.at[c], idxb)
            pltpu.sync_copy(cws.at[c], cwb)
            handles = [pltpu.async_copy(tall.at[idxb.at[t]], buf.at[t], sem)
                       for t in range(_NT)]
            for h in handles:
                h.wait()

            def row(k, carry2):
                wrow = cwb[k, :]
                for i in range(_CG // _LANES):
                    sl = pl.ds(i * _LANES, _LANES)
                    acc = wrow[0] * buf[0, k, sl]
                    for t in range(1, _NT):
                        acc = acc + wrow[t] * buf[t, k, sl]
                    obuf[k, sl] = acc
                return carry2

            lax.fori_loop(0, _K, row, 0)
            pltpu.sync_copy(obuf, out.at[pl.ds(wid * _PER_W + j * _K, _K)])
            return carry

        lax.fori_loop(0, _NCHUNK, chunk, 0)

    return sc_gather


_SC_GATHER = _sc_gather_fn()


def _build_tables(xs):
    parts = []
    for x in xs:
        b, cfull, h, w = x.shape
        t = x.reshape(b, N_GROUPS, _CG, h, w).transpose(0, 1, 3, 4, 2)
        parts.append(t.reshape(-1, _CG))
    return jnp.concatenate(parts, axis=0)


def _build_idx_weights(offset, xyzr):
    off = offset.reshape(B, N, N_GROUPS, IN_POINTS, 3)
    x = xyzr[..., 0][:, :, None, None]
    y = xyzr[..., 1][:, :, None, None]
    z = xyzr[..., 2][:, :, None, None]
    r = xyzr[..., 3][:, :, None, None]
    sx = 2.0 ** (z - 0.5 * r)
    sy = 2.0 ** (z + 0.5 * r)
    px = x + off[..., 0] * sx
    py = y + off[..., 1] * sy
    lvl = z + off[..., 2]

    grid = jnp.log2(jnp.asarray(STRIDES, jnp.float32))
    l2 = -jnp.abs(((lvl[..., None] - grid) ** 2) / 2.0)
    lw = jax.nn.softmax(l2, axis=-1)

    bg = (jnp.arange(B)[:, None, None, None] * N_GROUPS
          + jnp.arange(N_GROUPS)[None, None, :, None])

    idx_list, cw_list = [], []
    for i, stride in enumerate(STRIDES):
        hw = _SIZES[i]
        fx = px / stride - 0.5
        fy = py / stride - 0.5
        x0 = jnp.floor(fx)
        y0 = jnp.floor(fy)
        for dx, dy in ((0, 0), (1, 0), (0, 1), (1, 1)):
            xi = x0 + dx
            yi = y0 + dy
            valid = ((xi >= 0) & (xi <= hw - 1) & (yi >= 0) & (yi <= hw - 1))
            xc = jnp.clip(xi, 0, hw - 1).astype(jnp.int32)
            yc = jnp.clip(yi, 0, hw - 1).astype(jnp.int32)
            gid = _LVL_BASE[i] + (bg * hw + yc) * hw + xc
            wx = (x0 + 1.0 - fx) if dx == 0 else (fx - x0)
            wy = (y0 + 1.0 - fy) if dy == 0 else (fy - y0)
            cw = lw[..., i] * wx * wy * valid.astype(jnp.float32)
            idx_list.append(gid.reshape(-1))
            cw_list.append(cw.reshape(-1))
    idx = jnp.stack(idx_list, axis=0)
    cw = jnp.stack(cw_list, axis=0)
    idx = idx.reshape(_NT, _NW, _NCHUNK, _K).transpose(1, 2, 0, 3).reshape(
        _NW * _NCHUNK, _NT, _K)
    cw = cw.reshape(_NT, _NW, _NCHUNK, _K).transpose(1, 2, 3, 0).reshape(
        _NW * _NCHUNK, _K, _NT)
    return idx, cw


def _ln_kernel(q_ref, g_ref, b_ref, o_ref):
    q = q_ref[...]
    m = jnp.mean(q, axis=-1, keepdims=True)
    v = jnp.mean((q - m) ** 2, axis=-1, keepdims=True)
    o_ref[...] = (q - m) * jax.lax.rsqrt(v + 1e-5) * g_ref[...] + b_ref[...]


def _final_ln(q, ln_g, ln_b):
    Bq, Nq, C = q.shape
    out = pl.pallas_call(
        _ln_kernel,
        out_shape=jax.ShapeDtypeStruct((Bq * Nq, C), jnp.float32),
    )(q.reshape(Bq * Nq, C), ln_g.reshape(1, C), ln_b.reshape(1, C))
    return out.reshape(Bq, Nq, C)


def _layer_norm_2d(x, eps=1e-5):
    m = x.mean(axis=(-2, -1), keepdims=True)
    v = ((x - m) ** 2).mean(axis=(-2, -1), keepdims=True)
    return (x - m) / jnp.sqrt(v + eps)


def _adaptive_mixing(x, query, W_pg, b_pg, W_out, b_out):
    Bq, Nq, G, P, C = x.shape
    eff_out = FEAT_CH // N_GROUPS
    m_params = C * eff_out
    params = query @ W_pg + b_pg
    params = params.reshape(Bq * Nq, G, -1)
    out = x.reshape(Bq * Nq, G, P, C)
    M = params[..., :m_params].reshape(Bq * Nq, G, C, eff_out)
    S = params[..., m_params:].reshape(Bq * Nq, G, OUT_POINTS, IN_POINTS)
    out = jnp.matmul(out, M)
    out = jax.nn.relu(_layer_norm_2d(out))
    out = jnp.matmul(S, out)
    out = jax.nn.relu(_layer_norm_2d(out))
    out = out.reshape(Bq, Nq, -1)
    out = out @ W_out + b_out
    return query + out


def kernel(x0, x1, x2, x3, query_feat, query_roi, W_off, b_off, W_pg, b_pg, W_out, b_out, ln_g, ln_b):
    offset = query_feat @ W_off + b_off
    idx, cw = _build_idx_weights(offset, query_roi)
    tall = _build_tables([x0, x1, x2, x3])
    sampled = _SC_GATHER(tall, idx, cw).reshape(B, N, N_GROUPS, IN_POINTS, _CG)
    q = _adaptive_mixing(sampled, query_feat, W_pg, b_pg, W_out, b_out)
    return _final_ln(q, ln_g, ln_b)
